# baseline (speedup 1.0000x reference)
import functools

import jax
import jax.numpy as jnp
from jax import lax
from jax.experimental import pallas as pl
from jax.experimental.pallas import tpu as pltpu
from jax.experimental.pallas import tpu_sc as plsc

_info = plsc.get_sparse_core_info()
_NC, _NS, _NL = _info.num_cores, _info.num_subcores, _info.num_lanes
_NW = _NC * _NS

_RSTAGE = 200
_CHUNK = 400


def _copy_valid_lanes(src_ref, dst_ref, n_rows, d):
    def row(i, carry):
        for k in range(d // _NL):
            dst_ref[i, pl.ds(k * _NL, _NL)] = src_ref[i, pl.ds(k * _NL, _NL)]
        return carry

    lax.fori_loop(0, n_rows, row, 0)


def _stage_table(table):
    V, D = table.shape
    n_steps = V // _RSTAGE
    assert V % _RSTAGE == 0
    mesh = plsc.VectorSubcoreMesh(core_axis_name="c", subcore_axis_name="s")

    @functools.partial(
        pl.kernel,
        mesh=mesh,
        out_type=jax.ShapeDtypeStruct((V, 2 * D), jnp.float32),
        scratch_types=[
            pltpu.VMEM((_RSTAGE, D), jnp.float32),
            pltpu.VMEM((_RSTAGE, 2 * D), jnp.float32),
        ],
    )
    def k(table_hbm, staged_hbm, buf64_v, buf128_v):
        wid = lax.axis_index("s") * _NC + lax.axis_index("c")
        n_mine = (n_steps - wid + _NW - 1) // _NW

        def body(i, carry):
            base = (wid + i * _NW) * _RSTAGE
            pltpu.sync_copy(table_hbm.at[pl.ds(base, _RSTAGE)], buf64_v)
            _copy_valid_lanes(buf64_v, buf128_v, _RSTAGE, D)
            pltpu.sync_copy(buf128_v, staged_hbm.at[pl.ds(base, _RSTAGE)])
            return carry

        lax.fori_loop(0, n_mine, body, 0)

    return k(table)


def _gather_rows(staged, idx, D):
    B = idx.shape[0]
    b_per_w = B // _NW
    n_chunks = b_per_w // _CHUNK
    assert B % (_NW * _CHUNK) == 0
    mesh = plsc.VectorSubcoreMesh(core_axis_name="c", subcore_axis_name="s")

    @functools.partial(
        pl.kernel,
        mesh=mesh,
        out_type=jax.ShapeDtypeStruct((B, D), jnp.float32),
        scratch_types=[
            pltpu.VMEM((_CHUNK,), jnp.int32),
            pltpu.VMEM((_CHUNK, 2 * D), jnp.float32),
            pltpu.VMEM((_CHUNK, D), jnp.float32),
            pltpu.SemaphoreType.DMA,
        ],
    )
    def k(staged_hbm, idx_hbm, out_hbm, idx_v, rows_v, out_v, sem):
        wid = lax.axis_index("s") * _NC + lax.axis_index("c")
        base_w = wid * b_per_w

        def body(i, carry):
            base = base_w + i * _CHUNK
            pltpu.sync_copy(idx_hbm.at[pl.ds(base, _CHUNK)], idx_v)
            pltpu.async_copy(staged_hbm.at[idx_v], rows_v, sem).wait()
            _copy_valid_lanes(rows_v, out_v, _CHUNK, D)
            pltpu.sync_copy(out_v, out_hbm.at[pl.ds(base, _CHUNK)])
            return carry

        lax.fori_loop(0, n_chunks, body, 0)

    return k(staged, idx)


def kernel(input, W):
    D = W.shape[1]
    idx = input.reshape(-1).astype(jnp.int32)
    staged = _stage_table(W)
    out = _gather_rows(staged, idx, D)
    return out.reshape(input.shape + (D,))


# speedup vs baseline: 1.0041x; 1.0041x over previous
import functools

import jax
import jax.numpy as jnp
from jax import lax
from jax.experimental import pallas as pl
from jax.experimental.pallas import tpu as pltpu
from jax.experimental.pallas import tpu_sc as plsc

_info = plsc.get_sparse_core_info()
_NC, _NS, _NL = _info.num_cores, _info.num_subcores, _info.num_lanes
_NW = _NC * _NS

_CHUNK = 400


def _gather_rows(pairs, idx, D):
    B = idx.shape[0]
    b_per_w = B // _NW
    n_chunks = b_per_w // _CHUNK
    assert B % (_NW * _CHUNK) == 0
    mesh = plsc.VectorSubcoreMesh(core_axis_name="c", subcore_axis_name="s")

    @functools.partial(
        pl.kernel,
        mesh=mesh,
        out_type=jax.ShapeDtypeStruct((B, D), jnp.float32),
        scratch_types=[
            pltpu.VMEM((_CHUNK,), jnp.int32),
            pltpu.VMEM((_CHUNK,), jnp.int32),
            pltpu.VMEM((_CHUNK, 2 * D), jnp.float32),
            pltpu.VMEM((_CHUNK, D), jnp.float32),
            pltpu.SemaphoreType.DMA,
        ],
    )
    def k(pairs_hbm, idx_hbm, out_hbm, idx_v, u_v, rows_v, out_v, sem):
        wid = lax.axis_index("s") * _NC + lax.axis_index("c")
        base_w = wid * b_per_w

        def body(i, carry):
            base = base_w + i * _CHUNK
            pltpu.sync_copy(idx_hbm.at[pl.ds(base, _CHUNK)], idx_v)

            def halve(g, c):
                u_v[pl.ds(g * _NL, _NL)] = (
                    idx_v[pl.ds(g * _NL, _NL)] >> 1
                )
                return c

            lax.fori_loop(0, _CHUNK // _NL, halve, 0)
            pltpu.async_copy(pairs_hbm.at[u_v], rows_v, sem).wait()

            def select(g, c):
                pv = (idx_v[pl.ds(g * _NL, _NL)] & 1) * D
                base_r = g * _NL
                for j in range(_NL):
                    off = pv[j]
                    for kk in range(D // _NL):
                        out_v[base_r + j, pl.ds(kk * _NL, _NL)] = rows_v[
                            base_r + j, pl.ds(off + kk * _NL, _NL)
                        ]
                return c

            lax.fori_loop(0, _CHUNK // _NL, select, 0)
            pltpu.sync_copy(out_v, out_hbm.at[pl.ds(base, _CHUNK)])
            return carry

        lax.fori_loop(0, n_chunks, body, 0)

    return k(pairs, idx)


def kernel(input, W):
    V, D = W.shape
    idx = input.reshape(-1).astype(jnp.int32)
    pairs = W.reshape(V // 2, 2 * D)
    out = _gather_rows(pairs, idx, D)
    return out.reshape(input.shape + (D,))


# speedup vs baseline: 1.1982x; 1.1933x over previous
import functools

import jax
import jax.numpy as jnp
from jax import lax
from jax.experimental import pallas as pl
from jax.experimental.pallas import tpu as pltpu
from jax.experimental.pallas import tpu_sc as plsc

_info = plsc.get_sparse_core_info()
_NC, _NS, _NL = _info.num_cores, _info.num_subcores, _info.num_lanes
_NW = _NC * _NS

_CHUNK = 160


def _gather_rows(pairs, idx, out_shape3):
    B = idx.shape[0]
    D = pairs.shape[1] // 2
    b_per_w = B // _NW
    n_chunks = b_per_w // _CHUNK
    n_pairs = n_chunks // 2
    assert B % (_NW * _CHUNK) == 0 and n_chunks % 2 == 0
    mesh = plsc.VectorSubcoreMesh(core_axis_name="c", subcore_axis_name="s")

    @functools.partial(
        pl.kernel,
        mesh=mesh,
        out_type=jax.ShapeDtypeStruct(out_shape3, jnp.float32),
        scratch_types=[
            pltpu.VMEM((b_per_w,), jnp.int32),
            pltpu.VMEM((_CHUNK,), jnp.int32),
            pltpu.VMEM((_CHUNK,), jnp.int32),
            pltpu.VMEM((_CHUNK, 2 * D), jnp.float32),
            pltpu.VMEM((_CHUNK, 2 * D), jnp.float32),
            pltpu.VMEM((_CHUNK, D), jnp.float32),
            pltpu.VMEM((_CHUNK, D), jnp.float32),
            pltpu.SemaphoreType.DMA,
            pltpu.SemaphoreType.DMA,
            pltpu.SemaphoreType.DMA,
            pltpu.SemaphoreType.DMA,
        ],
    )
    def k(pairs_hbm, idx_hbm, out_hbm, idx_all, u0, u1, rows0, rows1,
          ov0, ov1, gsem0, gsem1, osem0, osem1):
        out2 = out_hbm.reshape(B, D)
        wid = lax.axis_index("s") * _NC + lax.axis_index("c")
        base_w = wid * b_per_w
        pltpu.sync_copy(idx_hbm.at[pl.ds(base_w, b_per_w)], idx_all)
        bufs = ((u0, rows0, ov0, gsem0, osem0),
                (u1, rows1, ov1, gsem1, osem1))

        def compute_u(u_v, c):
            def g16(g, cc):
                u_v[pl.ds(g * _NL, _NL)] = (
                    idx_all[pl.ds(c * _CHUNK + g * _NL, _NL)] >> 1
                )
                return cc

            lax.fori_loop(0, _CHUNK // _NL, g16, 0)

        def select(rows_v, out_v, c):
            def sel_g(g, cc):
                pv = (idx_all[pl.ds(c * _CHUNK + g * _NL, _NL)] & 1) * D
                for j in range(_NL):
                    off = pv[j]
                    r = g * _NL + j
                    for kk in range(D // _NL):
                        out_v[r, pl.ds(kk * _NL, _NL)] = rows_v[
                            r, pl.ds(off + kk * _NL, _NL)
                        ]
                return cc

            lax.fori_loop(0, _CHUNK // _NL, sel_g, 0)

        for c in (0, 1):
            u_v, rows_v, _, gsem, _ = bufs[c]
            compute_u(u_v, c)
            pltpu.async_copy(pairs_hbm.at[u_v], rows_v, gsem)

        def pair_body(t, carry):
            for j in range(2):
                u_v, rows_v, out_v, gsem, osem = bufs[j]
                c = 2 * t + j

                @pl.when(t > 0)
                def _():
                    pltpu.make_async_copy(
                        out_v, out2.at[pl.ds(0, _CHUNK)], osem
                    ).wait()

                pltpu.make_async_copy(
                    pairs_hbm.at[u_v], rows_v, gsem
                ).wait()
                select(rows_v, out_v, c)
                pltpu.async_copy(
                    out_v, out2.at[pl.ds(base_w + c * _CHUNK, _CHUNK)], osem
                )

                @pl.when(t < n_pairs - 1)
                def _():
                    compute_u(u_v, c + 2)
                    pltpu.async_copy(pairs_hbm.at[u_v], rows_v, gsem)

            return carry

        lax.fori_loop(0, n_pairs, pair_body, 0)
        for j in range(2):
            _, _, out_v, _, osem = bufs[j]
            pltpu.make_async_copy(
                out_v, out2.at[pl.ds(0, _CHUNK)], osem
            ).wait()

    return k(pairs, idx)


def kernel(input, W):
    V, D = W.shape
    idx = input.reshape(-1).astype(jnp.int32)
    pairs = W.reshape(V // 2, 2 * D)
    out = _gather_rows(pairs, idx, input.shape + (D,))
    return out.reshape(input.shape + (D,))


# speedup vs baseline: 1.2338x; 1.0298x over previous
import functools

import jax
import jax.numpy as jnp
from jax import lax
from jax.experimental import pallas as pl
from jax.experimental.pallas import tpu as pltpu
from jax.experimental.pallas import tpu_sc as plsc

_info = plsc.get_sparse_core_info()
_NC, _NS, _NL = _info.num_cores, _info.num_subcores, _info.num_lanes
_NW = _NC * _NS

_CHUNK = 256
_NBUF = 4


def _gather_rows(table, idx):
    V, D = table.shape
    B = idx.shape[0]
    b_per_w = B // _NW
    n_chunks = b_per_w // _CHUNK
    assert B % (_NW * _CHUNK) == 0 and n_chunks % _NBUF == 0
    mesh = plsc.VectorSubcoreMesh(core_axis_name="c", subcore_axis_name="s")

    @functools.partial(
        pl.kernel,
        mesh=mesh,
        compiler_params=pltpu.CompilerParams(use_tc_tiling_on_sc=False),
        out_type=jax.ShapeDtypeStruct((B, D), jnp.float32),
        scratch_types=(
            [pltpu.VMEM((b_per_w,), jnp.int32)]
            + [pltpu.VMEM((_CHUNK, D), jnp.float32) for _ in range(_NBUF)]
            + [pltpu.SemaphoreType.DMA for _ in range(2 * _NBUF)]
        ),
    )
    def k(table_hbm, idx_hbm, out_hbm, idx_all, r0, r1, r2, r3,
          g0, g1, g2, g3, o0, o1, o2, o3):
        rows = (r0, r1, r2, r3)
        gsem = (g0, g1, g2, g3)
        osem = (o0, o1, o2, o3)
        wid = lax.axis_index("s") * _NC + lax.axis_index("c")
        base_w = wid * b_per_w
        pltpu.sync_copy(idx_hbm.at[pl.ds(base_w, b_per_w)], idx_all)

        def issue_gather(c, b):
            pltpu.async_copy(
                table_hbm.at[idx_all.at[pl.ds(c * _CHUNK, _CHUNK)]],
                rows[b], gsem[b],
            )

        for c in range(2):
            issue_gather(c, c)

        def quad_body(t, carry):
            for j in range(_NBUF):
                c = _NBUF * t + j
                b2 = (j + 2) % _NBUF
                pltpu.make_async_copy(
                    table_hbm.at[idx_all.at[pl.ds(0, _CHUNK)]],
                    rows[j], gsem[j],
                ).wait()
                pltpu.async_copy(
                    rows[j],
                    out_hbm.at[pl.ds(base_w + c * _CHUNK, _CHUNK)],
                    osem[j],
                )

                @pl.when(c + 2 < n_chunks)
                def _():
                    @pl.when(c >= 2)
                    def _():
                        pltpu.make_async_copy(
                            rows[b2], out_hbm.at[pl.ds(0, _CHUNK)], osem[b2]
                        ).wait()

                    issue_gather(c + 2, b2)

            return carry

        lax.fori_loop(0, n_chunks // _NBUF, quad_body, 0)
        for j in range(_NBUF):
            pltpu.make_async_copy(
                rows[j], out_hbm.at[pl.ds(0, _CHUNK)], osem[j]
            ).wait()

    return k(table, idx)


def kernel(input, W):
    D = W.shape[1]
    idx = input.reshape(-1).astype(jnp.int32)
    out = _gather_rows(W, idx)
    return out.reshape(input.shape + (D,))


# speedup vs baseline: 1.2664x; 1.0264x over previous
import functools

import jax
import jax.numpy as jnp
from jax import lax
from jax.experimental import pallas as pl
from jax.experimental.pallas import tpu as pltpu
from jax.experimental.pallas import tpu_sc as plsc

_info = plsc.get_sparse_core_info()
_NC, _NS, _NL = _info.num_cores, _info.num_subcores, _info.num_lanes
_NW = _NC * _NS

_CHUNK = 256
_NBUF = 4


def _gather_rows(table, idx, idx2_shape):
    V, D = table.shape
    B = idx.shape[0]
    b_per_w = B // _NW
    n_chunks = b_per_w // _CHUNK
    assert B % (_NW * _CHUNK) == 0 and n_chunks % _NBUF == 0
    mesh = plsc.VectorSubcoreMesh(core_axis_name="c", subcore_axis_name="s")

    T, Bt = idx2_shape
    @functools.partial(
        pl.kernel,
        mesh=mesh,
        compiler_params=pltpu.CompilerParams(use_tc_tiling_on_sc=False),
        out_type=jax.ShapeDtypeStruct((T, Bt, D), jnp.float32),
        scratch_types=(
            [pltpu.VMEM((b_per_w,), jnp.int32)]
            + [pltpu.VMEM((_CHUNK, D), jnp.float32) for _ in range(_NBUF)]
            + [pltpu.SemaphoreType.DMA for _ in range(2 * _NBUF)]
        ),
    )
    def k(table_hbm, idx_hbm, out3_hbm, idx_all, r0, r1, r2, r3,
          g0, g1, g2, g3, o0, o1, o2, o3):
        rows = (r0, r1, r2, r3)
        gsem = (g0, g1, g2, g3)
        osem = (o0, o1, o2, o3)
        wid = lax.axis_index("s") * _NC + lax.axis_index("c")
        base_w = wid * b_per_w
        pltpu.sync_copy(idx_hbm.at[pl.ds(base_w, b_per_w)], idx_all)

        def issue_gather(c, b):
            pltpu.async_copy(
                table_hbm.at[idx_all.at[pl.ds(c * _CHUNK, _CHUNK)]],
                rows[b], gsem[b],
            )

        for c in range(2):
            issue_gather(c, c)

        def quad_body(t, carry):
            for j in range(_NBUF):
                c = _NBUF * t + j
                b2 = (j + 2) % _NBUF
                pltpu.make_async_copy(
                    table_hbm.at[idx_all.at[pl.ds(0, _CHUNK)]],
                    rows[j], gsem[j],
                ).wait()
                pos = base_w + c * _CHUNK
                pltpu.async_copy(
                    rows[j],
                    out3_hbm.at[pos // Bt, pl.ds(pos % Bt, _CHUNK)],
                    osem[j],
                )

                @pl.when(c + 2 < n_chunks)
                def _():
                    @pl.when(c >= 2)
                    def _():
                        pltpu.make_async_copy(
                            rows[b2], out3_hbm.at[0, pl.ds(0, _CHUNK)], osem[b2]
                        ).wait()

                    issue_gather(c + 2, b2)

            return carry

        lax.fori_loop(0, n_chunks // _NBUF, quad_body, 0)
        for j in range(_NBUF):
            pltpu.make_async_copy(
                rows[j], out3_hbm.at[0, pl.ds(0, _CHUNK)], osem[j]
            ).wait()

    return k(table, idx)


def kernel(input, W):
    Bv, T = input.shape
    idx = input.T.reshape(-1).astype(jnp.int32)
    out3 = _gather_rows(W, idx, (T, Bv))
    return out3.transpose(1, 0, 2)


# speedup vs baseline: 1.6100x; 1.2713x over previous
import functools

import jax
import jax.numpy as jnp
from jax import lax
from jax.experimental import pallas as pl
from jax.experimental.pallas import tpu as pltpu
from jax.experimental.pallas import tpu_sc as plsc
from jax.experimental.layout import Format, Layout, with_layout_constraint

_info = plsc.get_sparse_core_info()
_NC, _NS, _NL = _info.num_cores, _info.num_subcores, _info.num_lanes
_NW = _NC * _NS

_CHUNK = 256
_NBUF = 4


def _gather_rows(table, idx, idx2_shape):
    V, D = table.shape
    B = idx.shape[0]
    b_per_w = B // _NW
    n_chunks = b_per_w // _CHUNK
    assert B % (_NW * _CHUNK) == 0 and n_chunks % _NBUF == 0
    mesh = plsc.VectorSubcoreMesh(core_axis_name="c", subcore_axis_name="s")

    T, Bt = idx2_shape
    @functools.partial(
        pl.kernel,
        mesh=mesh,
        compiler_params=pltpu.CompilerParams(use_tc_tiling_on_sc=False),
        out_type=jax.ShapeDtypeStruct((T, Bt, D), jnp.float32),
        scratch_types=(
            [pltpu.VMEM((b_per_w,), jnp.int32)]
            + [pltpu.VMEM((_CHUNK, D), jnp.float32) for _ in range(_NBUF)]
            + [pltpu.SemaphoreType.DMA for _ in range(2 * _NBUF)]
        ),
    )
    def k(table_hbm, idx_hbm, out3_hbm, idx_all, r0, r1, r2, r3,
          g0, g1, g2, g3, o0, o1, o2, o3):
        rows = (r0, r1, r2, r3)
        gsem = (g0, g1, g2, g3)
        osem = (o0, o1, o2, o3)
        wid = lax.axis_index("s") * _NC + lax.axis_index("c")
        base_w = wid * b_per_w
        pltpu.sync_copy(idx_hbm.at[pl.ds(base_w, b_per_w)], idx_all)

        def issue_gather(c, b):
            pltpu.async_copy(
                table_hbm.at[idx_all.at[pl.ds(c * _CHUNK, _CHUNK)]],
                rows[b], gsem[b],
            )

        for c in range(2):
            issue_gather(c, c)

        def quad_body(t, carry):
            for j in range(_NBUF):
                c = _NBUF * t + j
                b2 = (j + 2) % _NBUF
                pltpu.make_async_copy(
                    table_hbm.at[idx_all.at[pl.ds(0, _CHUNK)]],
                    rows[j], gsem[j],
                ).wait()
                pos = base_w + c * _CHUNK
                pltpu.async_copy(
                    rows[j],
                    out3_hbm.at[pos // Bt, pl.ds(pos % Bt, _CHUNK)],
                    osem[j],
                )

                @pl.when(c + 2 < n_chunks)
                def _():
                    @pl.when(c >= 2)
                    def _():
                        pltpu.make_async_copy(
                            rows[b2], out3_hbm.at[0, pl.ds(0, _CHUNK)], osem[b2]
                        ).wait()

                    issue_gather(c + 2, b2)

            return carry

        lax.fori_loop(0, n_chunks // _NBUF, quad_body, 0)
        for j in range(_NBUF):
            pltpu.make_async_copy(
                rows[j], out3_hbm.at[0, pl.ds(0, _CHUNK)], osem[j]
            ).wait()

    return k(table, idx)


def kernel(input, W):
    Bv, T = input.shape
    idx = input.T.reshape(-1).astype(jnp.int32)
    W_sc = with_layout_constraint(
        W, Layout(major_to_minor=(0, 1), tiling=((8,),))
    )
    out3 = _gather_rows(W_sc, idx, (T, Bv))
    return out3.transpose(1, 0, 2)
